# Initial kernel scaffold; baseline (speedup 1.0000x reference)
import jax
import jax.numpy as jnp
from jax.experimental import pallas as pl


def kernel(input, mask):
    raise NotImplementedError("write your pallas kernel here")


# speedup vs baseline: 2.4704x; 2.4704x over previous
import functools
import jax
import jax.numpy as jnp
from jax import lax
from jax.experimental import pallas as pl
from jax.experimental.pallas import tpu as pltpu
from jax.experimental.pallas import tpu_sc as plsc

E = 320000
F = 128
S = 10000
EPS = 1e-9
NC = 2
NS = 16
NW = NC * NS
EPW = E // NW
NB = 80
NIT = EPW // NB
CL = 16
RPT = 624
TAIL = S - RPT * NS
ZR = 104

_MESH = plsc.VectorSubcoreMesh(core_axis_name="c", subcore_axis_name="s")
_PARAMS = pltpu.CompilerParams(use_tc_tiling_on_sc=False)


def _wid(c, s):
    return c * NS + s


@functools.partial(
    pl.kernel,
    out_type=(
        jax.ShapeDtypeStruct((NC, S, F), jnp.float32),
        jax.ShapeDtypeStruct((NC, S, CL), jnp.float32),
    ),
    mesh=_MESH,
    compiler_params=_PARAMS,
    scratch_types=[
        pltpu.VMEM_SHARED((S, F), jnp.float32),
        pltpu.VMEM_SHARED((S, CL), jnp.float32),
        pltpu.VMEM((NB, F), jnp.float32),
        pltpu.VMEM((NB,), jnp.int32),
        pltpu.VMEM((NB, CL), jnp.float32),
        pltpu.VMEM((ZR, F), jnp.float32),
        pltpu.VMEM((RPT + TAIL, CL), jnp.float32),
    ],
)
def _accumulate(x_hbm, seg_hbm, psum_hbm, pcnt_hbm,
                sp_sum, sp_cnt, xbuf, idxbuf, onesbuf, zbufx, zbufc):
    c = lax.axis_index("c")
    s = lax.axis_index("s")
    base = _wid(c, s) * EPW
    ZV = jnp.zeros((16,), jnp.float32)
    OV = jnp.ones((16,), jnp.float32)

    def fill_zx(r, _):
        for v in range(F // 16):
            zbufx[r, pl.ds(v * 16, 16)] = ZV
        return 0
    lax.fori_loop(0, ZR, fill_zx, 0)

    def fill_zc(r, _):
        zbufc[r, :] = ZV
        return 0
    lax.fori_loop(0, RPT + TAIL, fill_zc, 0)

    def fill_ones(r, _):
        onesbuf[r, :] = OV
        return 0
    lax.fori_loop(0, NB, fill_ones, 0)

    row0 = s * RPT
    for k in range(RPT // ZR):
        pltpu.sync_copy(zbufx, sp_sum.at[pl.ds(row0 + k * ZR, ZR)])
    pltpu.sync_copy(zbufc.at[pl.ds(0, RPT)], sp_cnt.at[pl.ds(row0, RPT)])

    @pl.when(s == NS - 1)
    def _zero_tail():
        pltpu.sync_copy(zbufx.at[pl.ds(0, TAIL)],
                        sp_sum.at[pl.ds(RPT * NS, TAIL)])
        pltpu.sync_copy(zbufc.at[pl.ds(0, TAIL)],
                        sp_cnt.at[pl.ds(RPT * NS, TAIL)])

    plsc.subcore_barrier()

    def body(j, _):
        off = base + j * NB
        pltpu.sync_copy(x_hbm.at[pl.ds(off, NB)], xbuf)
        pltpu.sync_copy(seg_hbm.at[pl.ds(off, NB)], idxbuf)
        pltpu.sync_copy(xbuf, sp_sum.at[idxbuf], add=True)
        pltpu.sync_copy(onesbuf, sp_cnt.at[idxbuf], add=True)
        return 0
    lax.fori_loop(0, NIT, body, 0)
    plsc.subcore_barrier()

    pltpu.sync_copy(sp_sum.at[pl.ds(row0, RPT)], psum_hbm.at[c, pl.ds(row0, RPT)])
    pltpu.sync_copy(sp_cnt.at[pl.ds(row0, RPT)], pcnt_hbm.at[c, pl.ds(row0, RPT)])

    @pl.when(s == NS - 1)
    def _copy_tail():
        pltpu.sync_copy(sp_sum.at[pl.ds(RPT * NS, TAIL)],
                        psum_hbm.at[c, pl.ds(RPT * NS, TAIL)])
        pltpu.sync_copy(sp_cnt.at[pl.ds(RPT * NS, TAIL)],
                        pcnt_hbm.at[c, pl.ds(RPT * NS, TAIL)])


NR = 200
NBLK = S // NR


@functools.partial(
    pl.kernel,
    out_type=jax.ShapeDtypeStruct((S, F), jnp.float32),
    mesh=_MESH,
    compiler_params=_PARAMS,
    scratch_types=[
        pltpu.VMEM((NR, F), jnp.float32),
        pltpu.VMEM((NR, F), jnp.float32),
        pltpu.VMEM((NR, CL), jnp.float32),
        pltpu.VMEM((NR, CL), jnp.float32),
    ],
)
def _normalize(psum_hbm, pcnt_hbm, pool_hbm, abuf, bbuf, cabuf, cbbuf):
    c = lax.axis_index("c")
    s = lax.axis_index("s")
    w = _wid(c, s)

    def do_block(blk):
        r0 = blk * NR
        pltpu.sync_copy(psum_hbm.at[0, pl.ds(r0, NR)], abuf)
        pltpu.sync_copy(psum_hbm.at[1, pl.ds(r0, NR)], bbuf)
        pltpu.sync_copy(pcnt_hbm.at[0, pl.ds(r0, NR)], cabuf)
        pltpu.sync_copy(pcnt_hbm.at[1, pl.ds(r0, NR)], cbbuf)

        def row(r, _):
            cnt = cabuf[r, :] + cbbuf[r, :] + EPS
            for v in range(F // 16):
                sl = pl.ds(v * 16, 16)
                abuf[r, sl] = (abuf[r, sl] + bbuf[r, sl]) / cnt
            return 0
        lax.fori_loop(0, NR, row, 0)
        pltpu.sync_copy(abuf, pool_hbm.at[pl.ds(r0, NR)])

    def outer(k, _):
        blk = w + k * NW

        @pl.when(blk < NBLK)
        def _():
            do_block(blk)
        return 0
    lax.fori_loop(0, (NBLK + NW - 1) // NW, outer, 0)


@functools.partial(
    pl.kernel,
    out_type=jax.ShapeDtypeStruct((E, F), jnp.float32),
    mesh=_MESH,
    compiler_params=_PARAMS,
    scratch_types=[
        pltpu.VMEM((NB, F), jnp.float32),
        pltpu.VMEM((NB,), jnp.int32),
        pltpu.SemaphoreType.DMA,
    ],
)
def _gather(pool_hbm, seg_hbm, out_hbm, rows, idxbuf, sem):
    c = lax.axis_index("c")
    s = lax.axis_index("s")
    base = _wid(c, s) * EPW

    def body(j, _):
        off = base + j * NB
        pltpu.sync_copy(seg_hbm.at[pl.ds(off, NB)], idxbuf)
        pltpu.async_copy(pool_hbm.at[idxbuf], rows, sem).wait()
        pltpu.sync_copy(rows, out_hbm.at[pl.ds(off, NB)])
        return 0
    lax.fori_loop(0, NIT, body, 0)


def kernel(input, mask):
    seg = mask[:, 0].astype(jnp.int32)
    psum, pcnt = _accumulate(input, seg)
    pooled = _normalize(psum, pcnt)
    return _gather(pooled, seg)


# speedup vs baseline: 3.9129x; 1.5839x over previous
import functools
import jax
import jax.numpy as jnp
from jax import lax
from jax.experimental import pallas as pl
from jax.experimental.pallas import tpu as pltpu
from jax.experimental.pallas import tpu_sc as plsc

E = 320000
F = 128
S = 10000
EPS = 1e-9
NC = 2
NS = 16
NW = NC * NS
NB = 80
NBLK = E // NB
NG = NB // 16
SPT = 320
SP = NW * SPT
NCH = E // 16
EPW = E // NW
NIT = EPW // NB
GRING = 5

_MESH = plsc.VectorSubcoreMesh(core_axis_name="c", subcore_axis_name="s")
_PARAMS = pltpu.CompilerParams(use_tc_tiling_on_sc=False,
                               needs_layout_passes=False)


@functools.partial(
    pl.kernel,
    out_type=jax.ShapeDtypeStruct((SP, F), jnp.float32),
    mesh=_MESH,
    compiler_params=_PARAMS,
    scratch_types=[
        pltpu.VMEM_SHARED((NS, SPT + 1, F), jnp.float32),
        pltpu.VMEM((NB, F), jnp.float32),
        pltpu.VMEM((SPT, F), jnp.float32),
        pltpu.VMEM((NB,), jnp.int32),
        pltpu.VMEM((NB,), jnp.int32),
        pltpu.VMEM((336,), jnp.float32),
        pltpu.VMEM((16,), jnp.int32),
        pltpu.VMEM((16,), jnp.int32),
        pltpu.VMEM((8, F), jnp.float32),
        pltpu.SemaphoreType.DMA,
    ],
)
def _pool(x_hbm, seg_hbm, pool_hbm,
          sp_acc, xbuf, normbuf, segrow, idxbuf, cntl, bsbuf, tmp16, zbuf,
          zsem):
    c = lax.axis_index("c")
    s = lax.axis_index("s")
    w = c * NS + s
    segbase = w * SPT
    ZV = jnp.zeros((16,), jnp.float32)
    IOTA = lax.iota(jnp.int32, 16)

    def fz(r, _):
        for v in range(F // 16):
            zbuf[r, pl.ds(v * 16, 16)] = ZV
        return 0
    lax.fori_loop(0, 8, fz, 0)

    def fzc(k, _):
        cntl[pl.ds(k * 16, 16)] = ZV
        return 0
    lax.fori_loop(0, 336 // 16, fzc, 0)

    for k in range((SPT + 1) // 8):
        pltpu.async_copy(zbuf, sp_acc.at[s, pl.ds(k * 8, 8)], zsem)
    for k in range((SPT + 1) // 8):
        pltpu.make_async_copy(zbuf, sp_acc.at[s, pl.ds(k * 8, 8)],
                              zsem).wait()
    pltpu.sync_copy(zbuf.at[pl.ds(0, 1)], sp_acc.at[s, pl.ds(SPT, 1)])

    def lower_bound(B):
        def probe(_, lohi):
            lo, hi = lohi
            mid = lax.div(lo + hi, jnp.int32(2))
            pltpu.sync_copy(seg_hbm.at[pl.ds(mid * 16, 16)], bsbuf)
            vmin = lax.reduce_min(bsbuf[:], (0,))
            go_left = vmin >= B
            return (jnp.where(go_left, lo, mid), jnp.where(go_left, mid, hi))
        lo, _ = lax.fori_loop(0, 15, probe,
                              (jnp.int32(0), jnp.int32(NCH)))
        pltpu.sync_copy(seg_hbm.at[pl.ds(lo * 16, 16)], bsbuf)
        nlt = lax.reduce_sum((bsbuf[:] < B).astype(jnp.int32), (0,))
        return lo * 16 + nlt

    e_lo = lower_bound(segbase)
    e_hi = lower_bound(segbase + SPT)
    jb_lo = lax.div(e_lo, jnp.int32(NB))
    jb_hi = lax.div(e_hi + NB - 1, jnp.int32(NB))

    def shift_right_i32(x, fill):
        tmp16[:] = x
        y = plsc.load_gather(tmp16, [jnp.maximum(IOTA - 1, 0)])
        return jnp.where(IOTA == 0, fill, y)

    def shift_left_i32(x, fill):
        tmp16[:] = x
        y = plsc.load_gather(tmp16, [jnp.minimum(IOTA + 1, 15)])
        return jnp.where(IOTA == 15, fill, y)

    def body(jb, _):
        off = jb * NB
        pltpu.sync_copy(x_hbm.at[pl.ds(off, NB)], xbuf)
        pltpu.sync_copy(seg_hbm.at[pl.ds(off, NB)], segrow)
        for k in range(NG):
            iv = segrow[pl.ds(k * 16, 16)]
            lidx = iv - segbase
            inr = (lidx >= 0) & (lidx < SPT)
            lidxc = jnp.where(inr, lidx, SPT)
            idxbuf[pl.ds(k * 16, 16)] = lidxc
            prev = shift_right_i32(iv, -1)
            m = iv != prev
            sp = jnp.where(m, IOTA, 16)
            z = lax.rev(plsc.cummax(lax.rev(-sp, (0,))), (0,))
            ns = -shift_left_i32(z, -16)
            runlen = (ns - IOTA).astype(jnp.float32)
            plsc.addupdate_scatter(cntl, [lidxc], runlen, mask=m & inr)
        pltpu.sync_copy(xbuf, sp_acc.at[s].at[idxbuf], add=True)
        return 0
    lax.fori_loop(jb_lo, jb_hi, body, 0)

    pltpu.sync_copy(sp_acc.at[s, pl.ds(0, SPT)], normbuf)

    def rown(r, _):
        cv = plsc.load_gather(cntl, [jnp.broadcast_to(r, (16,))]) + EPS
        for v in range(F // 16):
            sl = pl.ds(v * 16, 16)
            normbuf[r, sl] = normbuf[r, sl] / cv
        return 0
    lax.fori_loop(0, SPT, rown, 0)
    pltpu.sync_copy(normbuf, pool_hbm.at[pl.ds(w * SPT, SPT)])


@functools.partial(
    pl.kernel,
    out_type=jax.ShapeDtypeStruct((E, F), jnp.float32),
    mesh=_MESH,
    compiler_params=_PARAMS,
    scratch_types=[pltpu.VMEM((NIT, NB), jnp.int32)]
    + [pltpu.VMEM((NB, F), jnp.float32)] * GRING
    + [pltpu.SemaphoreType.DMA] * (2 * GRING),
)
def _gather(pool_hbm, seg3_hbm, out_hbm, segbuf, *rest):
    rows = rest[:GRING]
    gsems = rest[GRING:2 * GRING]
    wsems = rest[2 * GRING:3 * GRING]
    c = lax.axis_index("c")
    s = lax.axis_index("s")
    base = (c * NS + s) * EPW

    pltpu.sync_copy(seg3_hbm.at[c * NS + s], segbuf)
    for b in range(GRING):
        pltpu.async_copy(pool_hbm.at[segbuf.at[b]], rows[b], gsems[b])

    def wait_g(b, j):
        pltpu.make_async_copy(pool_hbm.at[segbuf.at[j]], rows[b],
                              gsems[b]).wait()

    def wait_w(b, j):
        pltpu.make_async_copy(rows[b], out_hbm.at[pl.ds(base + j * NB, NB)],
                              wsems[b]).wait()

    def outer(g, _):
        for b in range(GRING):
            j = g * GRING + b
            wait_g(b, j)
            pltpu.async_copy(rows[b], out_hbm.at[pl.ds(base + j * NB, NB)],
                             wsems[b])
            jn = j + GRING

            @pl.when(jn < NIT)
            def _refill():
                wait_w(b, j)
                pltpu.async_copy(pool_hbm.at[segbuf.at[jn]], rows[b],
                                 gsems[b])
        return 0
    lax.fori_loop(0, NIT // GRING, outer, 0)
    for b in range(GRING):
        wait_w(b, 0)


def kernel(input, mask):
    seg = mask[:, 0].astype(jnp.int32)
    pooled = _pool(input, seg)
    return _gather(pooled, seg.reshape(NW, NIT, NB))


# speedup vs baseline: 5.5225x; 1.4114x over previous
import functools
import jax
import jax.numpy as jnp
from jax import lax
from jax.experimental import pallas as pl
from jax.experimental.pallas import tpu as pltpu
from jax.experimental.pallas import tpu_sc as plsc

E = 320000
F = 128
S = 10000
EPS = 1e-9
NC = 2
NS = 16
NW = NC * NS
NB = 80
NBLK = E // NB
NG = NB // 16
SPT = 320
SP = NW * SPT
NCH = E // 16
EPW = E // NW
GNB = 125
NIT = EPW // GNB
GRING = 5

_MESH = plsc.VectorSubcoreMesh(core_axis_name="c", subcore_axis_name="s")
_PARAMS = pltpu.CompilerParams(use_tc_tiling_on_sc=False,
                               needs_layout_passes=False)


@functools.partial(
    pl.kernel,
    out_type=jax.ShapeDtypeStruct((SP, F), jnp.float32),
    mesh=_MESH,
    compiler_params=_PARAMS,
    scratch_types=[
        pltpu.VMEM_SHARED((NS, SPT + 1, F), jnp.float32),
        pltpu.VMEM((NB, F), jnp.float32),
        pltpu.VMEM((NB, F), jnp.float32),
        pltpu.VMEM((SPT, F), jnp.float32),
        pltpu.VMEM((NB,), jnp.int32),
        pltpu.VMEM((NB,), jnp.int32),
        pltpu.VMEM((NB,), jnp.int32),
        pltpu.VMEM((336,), jnp.float32),
        pltpu.VMEM((16,), jnp.int32),
        pltpu.VMEM((16,), jnp.int32),
        pltpu.VMEM((8, F), jnp.float32),
        pltpu.SemaphoreType.DMA,
        pltpu.SemaphoreType.DMA,
        pltpu.SemaphoreType.DMA,
    ],
)
def _pool(x_hbm, seg_hbm, pool_hbm,
          sp_acc, xbuf0, xbuf1, normbuf, segrow0, segrow1, idxbuf, cntl,
          bsbuf, tmp16, zbuf, zsem, psem0, psem1):
    xbufs = (xbuf0, xbuf1)
    segrows = (segrow0, segrow1)
    psems = (psem0, psem1)
    c = lax.axis_index("c")
    s = lax.axis_index("s")
    w = c * NS + s
    segbase = w * SPT
    ZV = jnp.zeros((16,), jnp.float32)
    IOTA = lax.iota(jnp.int32, 16)

    def fz(r, _):
        for v in range(F // 16):
            zbuf[r, pl.ds(v * 16, 16)] = ZV
        return 0
    lax.fori_loop(0, 8, fz, 0)

    def fzc(k, _):
        cntl[pl.ds(k * 16, 16)] = ZV
        return 0
    lax.fori_loop(0, 336 // 16, fzc, 0)

    for k in range((SPT + 1) // 8):
        pltpu.async_copy(zbuf, sp_acc.at[s, pl.ds(k * 8, 8)], zsem)
    for k in range((SPT + 1) // 8):
        pltpu.make_async_copy(zbuf, sp_acc.at[s, pl.ds(k * 8, 8)],
                              zsem).wait()
    pltpu.sync_copy(zbuf.at[pl.ds(0, 1)], sp_acc.at[s, pl.ds(SPT, 1)])

    def lower_bound(B):
        def probe(_, lohi):
            lo, hi = lohi
            mid = lax.div(lo + hi, jnp.int32(2))
            pltpu.sync_copy(seg_hbm.at[pl.ds(mid * 16, 16)], bsbuf)
            vmin = lax.reduce_min(bsbuf[:], (0,))
            go_left = vmin >= B
            return (jnp.where(go_left, lo, mid), jnp.where(go_left, mid, hi))
        lo, _ = lax.fori_loop(0, 15, probe,
                              (jnp.int32(0), jnp.int32(NCH)))
        pltpu.sync_copy(seg_hbm.at[pl.ds(lo * 16, 16)], bsbuf)
        nlt = lax.reduce_sum((bsbuf[:] < B).astype(jnp.int32), (0,))
        return lo * 16 + nlt

    e_lo = lower_bound(segbase)
    e_hi = lower_bound(segbase + SPT)
    jb_lo = lax.div(e_lo, jnp.int32(NB))
    jb_hi = lax.div(e_hi + NB - 1, jnp.int32(NB))

    def shift_right_i32(x, fill):
        tmp16[:] = x
        y = plsc.load_gather(tmp16, [jnp.maximum(IOTA - 1, 0)])
        return jnp.where(IOTA == 0, fill, y)

    def shift_left_i32(x, fill):
        tmp16[:] = x
        y = plsc.load_gather(tmp16, [jnp.minimum(IOTA + 1, 15)])
        return jnp.where(IOTA == 15, fill, y)

    def issue(jb, b):
        off = jb * NB
        pltpu.async_copy(x_hbm.at[pl.ds(off, NB)], xbufs[b], psems[b])
        pltpu.async_copy(seg_hbm.at[pl.ds(off, NB)], segrows[b], psems[b])

    def wait_pf(jb, b):
        off = jb * NB
        pltpu.make_async_copy(x_hbm.at[pl.ds(off, NB)], xbufs[b],
                              psems[b]).wait()
        pltpu.make_async_copy(seg_hbm.at[pl.ds(off, NB)], segrows[b],
                              psems[b]).wait()

    for b in range(2):
        @pl.when(jb_lo + b < jb_hi)
        def _prime():
            issue(jb_lo + b, b)

    def pair(g, _):
        for b in range(2):
            jb = jb_lo + g * 2 + b

            @pl.when(jb < jb_hi)
            def _step():
                wait_pf(jb, b)
                for k in range(NG):
                    iv = segrows[b][pl.ds(k * 16, 16)]
                    lidx = iv - segbase
                    inr = (lidx >= 0) & (lidx < SPT)
                    lidxc = jnp.where(inr, lidx, SPT)
                    idxbuf[pl.ds(k * 16, 16)] = lidxc
                    prev = shift_right_i32(iv, -1)
                    m = iv != prev
                    sp = jnp.where(m, IOTA, 16)
                    z = lax.rev(plsc.cummax(lax.rev(-sp, (0,))), (0,))
                    ns = -shift_left_i32(z, -16)
                    runlen = (ns - IOTA).astype(jnp.float32)
                    plsc.addupdate_scatter(cntl, [lidxc], runlen,
                                           mask=m & inr)

                pltpu.sync_copy(xbufs[b], sp_acc.at[s].at[idxbuf], add=True)

                @pl.when(jb + 2 < jb_hi)
                def _refill():
                    issue(jb + 2, b)
        return 0
    npairs = lax.div(jb_hi - jb_lo + 1, jnp.int32(2))
    lax.fori_loop(0, npairs, pair, 0)

    pltpu.sync_copy(sp_acc.at[s, pl.ds(0, SPT)], normbuf)

    def rown(r, _):
        cv = plsc.load_gather(cntl, [jnp.broadcast_to(r, (16,))]) + EPS
        for v in range(F // 16):
            sl = pl.ds(v * 16, 16)
            normbuf[r, sl] = normbuf[r, sl] / cv
        return 0
    lax.fori_loop(0, SPT, rown, 0)
    pltpu.sync_copy(normbuf, pool_hbm.at[pl.ds(w * SPT, SPT)])


@functools.partial(
    pl.kernel,
    out_type=jax.ShapeDtypeStruct((E, F), jnp.float32),
    mesh=_MESH,
    compiler_params=_PARAMS,
    scratch_types=[pltpu.VMEM((NIT, GNB), jnp.int32)]
    + [pltpu.VMEM((GNB, F), jnp.float32)] * GRING
    + [pltpu.SemaphoreType.DMA] * (2 * GRING),
)
def _gather(pool_hbm, seg3_hbm, out_hbm, segbuf, *rest):
    rows = rest[:GRING]
    gsems = rest[GRING:2 * GRING]
    wsems = rest[2 * GRING:3 * GRING]
    c = lax.axis_index("c")
    s = lax.axis_index("s")
    base = (c * NS + s) * EPW

    pltpu.sync_copy(seg3_hbm.at[c * NS + s], segbuf)
    for b in range(GRING):
        pltpu.async_copy(pool_hbm.at[segbuf.at[b]], rows[b], gsems[b])

    def wait_g(b, j):
        pltpu.make_async_copy(pool_hbm.at[segbuf.at[j]], rows[b],
                              gsems[b]).wait()

    def wait_w(b, j):
        pltpu.make_async_copy(rows[b],
                              out_hbm.at[pl.ds(base + j * GNB, GNB)],
                              wsems[b]).wait()

    def outer(g, _):
        for b in range(GRING):
            j = g * GRING + b
            wait_g(b, j)
            pltpu.async_copy(rows[b],
                             out_hbm.at[pl.ds(base + j * GNB, GNB)],
                             wsems[b])
            jn = j + GRING

            @pl.when(jn < NIT)
            def _refill():
                wait_w(b, j)
                pltpu.async_copy(pool_hbm.at[segbuf.at[jn]], rows[b],
                                 gsems[b])
        return 0
    lax.fori_loop(0, NIT // GRING, outer, 0)
    for b in range(GRING):
        wait_w(b, 0)


def kernel(input, mask):
    seg = mask[:, 0].astype(jnp.int32)
    pooled = _pool(input, seg)
    return _gather(pooled, seg.reshape(NW, NIT, GNB))
